# baseline (speedup 1.0000x reference)
import functools

import jax
import jax.numpy as jnp
from jax import lax
from jax.experimental import pallas as pl
from jax.experimental.pallas import tpu as pltpu
from jax.experimental.pallas import tpu_sc as plsc

N = 5000
NCAND = 2000
NW = 32
RPW = 160
NPAD = NW * RPW
ROW = NCAND * 5
SROW = NCAND * 2
NCHUNK = (RPW + 15) // 16

K = 0.055

_COS = (0.9999999999999956, -4.934802200543692, 4.058712126381724,
        -1.3352627683856102, 0.23533062721525902, -0.025806879372614583,
        0.0019295464417837132, -0.00010459814866408124, 4.268378688720986e-06)


def _rsqrt(x):
    i = lax.bitcast_convert_type(x, jnp.int32)
    i = jnp.int32(0x5F3759DF) - lax.shift_right_arithmetic(i, 1)
    y = lax.bitcast_convert_type(i, jnp.float32)
    for _ in range(3):
        y = y * (jnp.float32(1.5) - jnp.float32(0.5) * x * y * y)
    return y


def _coswin(v):
    acc = jnp.full_like(v, jnp.float32(_COS[-1]))
    for c in _COS[-2::-1]:
        acc = acc * v + jnp.float32(c)
    return acc


def _make_sc_call():
    mesh = plsc.VectorSubcoreMesh(core_axis_name="c", subcore_axis_name="s")

    @functools.partial(
        pl.kernel,
        mesh=mesh,
        compiler_params=pltpu.CompilerParams(needs_layout_passes=False),
        out_type=[
            jax.ShapeDtypeStruct((NPAD * 4,), jnp.float32),
            jax.ShapeDtypeStruct((NPAD,), jnp.float32),
        ],
        scratch_types=[
            pltpu.VMEM((RPW * 4,), jnp.float32),
            pltpu.VMEM((ROW,), jnp.float32),
            pltpu.VMEM((SROW,), jnp.float32),
            pltpu.VMEM((NCHUNK * 16,), jnp.float32),
            pltpu.VMEM((NCHUNK * 16,), jnp.float32),
            pltpu.VMEM((NCHUNK * 16,), jnp.float32),
            pltpu.VMEM((NCHUNK * 16,), jnp.float32),
            pltpu.VMEM((NCHUNK * 16,), jnp.float32),
            pltpu.VMEM((NCHUNK * 16,), jnp.float32),
            pltpu.VMEM((RPW * 4,), jnp.float32),
            pltpu.VMEM((RPW,), jnp.float32),
        ],
    )
    def trnms(rois_hbm, rpn_hbm, sc_hbm, out1_hbm, out2_hbm,
              rois_v, row_v, srow_v, cx, cy, cs, cis, cal, cga,
              o1v, o2v):
        wid = lax.axis_index("s") * 2 + lax.axis_index("c")
        base = wid * RPW
        cnt = jnp.minimum(jnp.int32(RPW), jnp.int32(N) - base)

        iota = lax.iota(jnp.int32, 16)
        iota5 = iota * 5
        iota2 = iota * 2
        zeros = jnp.zeros((16,), jnp.int32)

        pltpu.sync_copy(rois_hbm.at[pl.ds(base * 4, RPW * 4)], rois_v)

        for t in range(NCHUNK):
            li = jnp.minimum(iota + (16 * t), jnp.int32(RPW - 1)) * 4
            c0 = plsc.load_gather(rois_v, [li])
            c1 = plsc.load_gather(rois_v, [li + 1])
            c2 = plsc.load_gather(rois_v, [li + 2])
            c3 = plsc.load_gather(rois_v, [li + 3])
            x = (c0 + c2) * jnp.float32(0.5)
            y = (c1 + c3) * jnp.float32(0.5)
            w = jnp.abs(c0 - c2) + jnp.float32(1e-4)
            h = jnp.abs(c1 - c3) + jnp.float32(1e-4)
            p = (w + h) * jnp.float32(0.5)
            q = (w + p) * (h + p)
            rq = _rsqrt(q)
            s = q * rq
            r = w / h
            r_max = jnp.maximum(r, jnp.float32(1.0) / r)
            sl = pl.ds(16 * t, 16)
            cx[sl] = x
            cy[sl] = y
            cs[sl] = s
            cis[sl] = rq
            cal[sl] = jnp.float32(-K) * r_max
            cga[sl] = rq * rq * jnp.float32(0.25)

        def per_roi(i, carry):
            rid = base + i
            pltpu.sync_copy(rpn_hbm.at[pl.ds(rid * ROW, ROW)], row_v)
            pltpu.sync_copy(sc_hbm.at[pl.ds(rid * SROW, SROW)], srow_v)

            ii = zeros + i
            x = plsc.load_gather(cx, [ii])
            y = plsc.load_gather(cy, [ii])
            s = plsc.load_gather(cs, [ii])
            inv_s = plsc.load_gather(cis, [ii])
            alpha = plsc.load_gather(cal, [ii])
            gamma = plsc.load_gather(cga, [ii])

            def cand_step(jj, st):
                bestv, besti = st
                b80 = jj * 80
                c0 = plsc.load_gather(row_v, [iota5 + (b80 + 1)])
                c1 = plsc.load_gather(row_v, [iota5 + (b80 + 2)])
                c2 = plsc.load_gather(row_v, [iota5 + (b80 + 3)])
                c3 = plsc.load_gather(row_v, [iota5 + (b80 + 4)])
                s1 = plsc.load_gather(srow_v, [iota2 + (jj * 32 + 1)])

                x_ = (c0 + c2) * jnp.float32(0.5)
                y_ = (c1 + c3) * jnp.float32(0.5)
                w_ = jnp.abs(c0 - c2) + jnp.float32(1e-4)
                h_ = jnp.abs(c1 - c3) + jnp.float32(1e-4)
                p_ = (w_ + h_) * jnp.float32(0.5)
                a_ = (w_ + p_) * (h_ + p_)
                ra = _rsqrt(a_)
                s_max = jnp.maximum(s * ra, a_ * ra * inv_s)
                pen = jnp.exp(alpha * s_max + jnp.float32(K))

                dx = x - x_
                dy = y - y_
                v = (dx * dx + dy * dy) * gamma
                han42 = jnp.float32(0.21) * _coswin(v) + jnp.float32(0.21)
                han42 = jnp.where(v > jnp.float32(1.0), jnp.float32(0.0), han42)

                score = s1 * pen + han42
                jvec = iota + jj * 16
                upd = score > bestv
                bestv = jnp.where(upd, score, bestv)
                besti = jnp.where(upd, jvec, besti)
                return bestv, besti

            bestv, besti = lax.fori_loop(
                0, NCAND // 16, cand_step,
                (jnp.full((16,), -jnp.inf, jnp.float32),
                 jnp.zeros((16,), jnp.int32)))

            m = jnp.max(bestv)
            cand = jnp.where(bestv == m, besti, jnp.int32(1 << 30))
            minj = jnp.min(cand)

            sel_idx = jnp.minimum(minj * 5 + 1 + iota, jnp.int32(ROW - 1))
            vals = plsc.load_gather(row_v, [sel_idx])
            plsc.store_scatter(o1v, [zeros + i * 4 + iota], vals, mask=iota < 4)
            sval = plsc.load_gather(
                srow_v, [jnp.minimum(zeros + (minj * 2 + 1),
                                     jnp.int32(SROW - 1))])
            plsc.store_scatter(o2v, [zeros + i], sval, mask=iota < 1)
            return carry

        lax.fori_loop(0, cnt, per_roi, jnp.int32(0))

        pltpu.sync_copy(o1v, out1_hbm.at[pl.ds(base * 4, RPW * 4)])
        pltpu.sync_copy(o2v, out2_hbm.at[pl.ds(base, RPW)])

    return trnms


_sc_trnms = _make_sc_call()


@jax.jit
def kernel(rois, rpn_rois, scores):
    rois_p = jnp.pad(rois, ((0, NPAD - N), (0, 0))).reshape(-1)
    rpn_flat = rpn_rois.reshape(-1)
    sc_flat = scores.reshape(-1)
    out1, out2 = _sc_trnms(rois_p, rpn_flat, sc_flat)
    return out1.reshape(NPAD, 4)[:N], out2.reshape(NPAD, 1)[:N]


# speedup vs baseline: 52.9990x; 52.9990x over previous
import functools

import jax
import jax.numpy as jnp
from jax import lax
from jax.experimental import pallas as pl
from jax.experimental.pallas import tpu as pltpu
from jax.experimental.pallas import tpu_sc as plsc

N = 5000
NCAND = 2000
NW = 32
RPW = 160
NPAD = NW * RPW
ROW = NCAND * 5
SROW = NCAND * 2
NCHUNK = (RPW + 15) // 16

K = 0.055

_COS = (0.9999999999999956, -4.934802200543692, 4.058712126381724,
        -1.3352627683856102, 0.23533062721525902, -0.025806879372614583,
        0.0019295464417837132, -0.00010459814866408124, 4.268378688720986e-06)


def _rsqrt(x):
    i = lax.bitcast_convert_type(x, jnp.int32)
    i = jnp.int32(0x5F3759DF) - lax.shift_right_arithmetic(i, 1)
    y = lax.bitcast_convert_type(i, jnp.float32)
    for _ in range(3):
        y = y * (jnp.float32(1.5) - jnp.float32(0.5) * x * y * y)
    return y


def _coswin(v):
    acc = jnp.full_like(v, jnp.float32(_COS[-1]))
    for c in _COS[-2::-1]:
        acc = acc * v + jnp.float32(c)
    return acc


def _make_sc_call():
    mesh = plsc.VectorSubcoreMesh(core_axis_name="c", subcore_axis_name="s")

    @functools.partial(
        pl.kernel,
        mesh=mesh,
        compiler_params=pltpu.CompilerParams(needs_layout_passes=False),
        out_type=[
            jax.ShapeDtypeStruct((NPAD * 4,), jnp.float32),
            jax.ShapeDtypeStruct((NPAD,), jnp.float32),
        ],
        scratch_types=[
            pltpu.VMEM((RPW * 4,), jnp.float32),
            pltpu.VMEM((ROW,), jnp.float32),
            pltpu.VMEM((SROW,), jnp.float32),
            pltpu.VMEM((NCHUNK * 16,), jnp.float32),
            pltpu.VMEM((NCHUNK * 16,), jnp.float32),
            pltpu.VMEM((NCHUNK * 16,), jnp.float32),
            pltpu.VMEM((NCHUNK * 16,), jnp.float32),
            pltpu.VMEM((NCHUNK * 16,), jnp.float32),
            pltpu.VMEM((NCHUNK * 16,), jnp.float32),
            pltpu.VMEM((RPW * 4,), jnp.float32),
            pltpu.VMEM((RPW,), jnp.float32),
        ],
    )
    def trnms(rois_hbm, rpn_hbm, sc_hbm, out1_hbm, out2_hbm,
              rois_v, row_v, srow_v, cx, cy, cs, cis, cal, cga,
              o1v, o2v):
        wid = lax.axis_index("s") * 2 + lax.axis_index("c")
        base = wid * RPW
        cnt = jnp.minimum(jnp.int32(RPW), jnp.int32(N) - base)

        iota = lax.iota(jnp.int32, 16)
        iota5 = iota * 5
        iota2 = iota * 2
        zeros = jnp.zeros((16,), jnp.int32)

        pltpu.sync_copy(rois_hbm.at[pl.ds(base * 4, RPW * 4)], rois_v)

        for t in range(NCHUNK):
            li = jnp.minimum(iota + (16 * t), jnp.int32(RPW - 1)) * 4
            c0 = plsc.load_gather(rois_v, [li])
            c1 = plsc.load_gather(rois_v, [li + 1])
            c2 = plsc.load_gather(rois_v, [li + 2])
            c3 = plsc.load_gather(rois_v, [li + 3])
            x = (c0 + c2) * jnp.float32(0.5)
            y = (c1 + c3) * jnp.float32(0.5)
            w = jnp.abs(c0 - c2) + jnp.float32(1e-4)
            h = jnp.abs(c1 - c3) + jnp.float32(1e-4)
            p = (w + h) * jnp.float32(0.5)
            q = (w + p) * (h + p)
            rq = _rsqrt(q)
            s = q * rq
            r = w / h
            r_max = jnp.maximum(r, jnp.float32(1.0) / r)
            sl = pl.ds(16 * t, 16)
            cx[sl] = x
            cy[sl] = y
            cs[sl] = s
            cis[sl] = rq
            cal[sl] = jnp.float32(-K) * r_max
            cga[sl] = rq * rq * jnp.float32(0.25)

        def per_roi(i, carry):
            rid = base + i
            pltpu.sync_copy(rpn_hbm.at[pl.ds(rid * ROW, ROW)], row_v)
            pltpu.sync_copy(sc_hbm.at[pl.ds(rid * SROW, SROW)], srow_v)

            ii = zeros + i
            x = plsc.load_gather(cx, [ii])
            y = plsc.load_gather(cy, [ii])
            s = plsc.load_gather(cs, [ii])
            inv_s = plsc.load_gather(cis, [ii])
            alpha = plsc.load_gather(cal, [ii])
            gamma = plsc.load_gather(cga, [ii])

            def cand_step(jj, st):
                bestv, besti = st
                b80 = jj * 80
                c0 = plsc.load_gather(row_v, [iota5 + (b80 + 1)])
                c1 = plsc.load_gather(row_v, [iota5 + (b80 + 2)])
                c2 = plsc.load_gather(row_v, [iota5 + (b80 + 3)])
                c3 = plsc.load_gather(row_v, [iota5 + (b80 + 4)])
                s1 = plsc.load_gather(srow_v, [iota2 + (jj * 32 + 1)])

                x_ = (c0 + c2) * jnp.float32(0.5)
                y_ = (c1 + c3) * jnp.float32(0.5)
                w_ = jnp.abs(c0 - c2) + jnp.float32(1e-4)
                h_ = jnp.abs(c1 - c3) + jnp.float32(1e-4)
                p_ = (w_ + h_) * jnp.float32(0.5)
                a_ = (w_ + p_) * (h_ + p_)
                ra = _rsqrt(a_)
                s_max = jnp.maximum(s * ra, a_ * ra * inv_s)
                pen = jnp.exp(alpha * s_max + jnp.float32(K))

                dx = x - x_
                dy = y - y_
                v = (dx * dx + dy * dy) * gamma
                han42 = jnp.float32(0.21) * _coswin(v) + jnp.float32(0.21)
                han42 = jnp.where(v > jnp.float32(1.0), jnp.float32(0.0), han42)

                score = s1 * pen + han42
                jvec = iota + jj * 16
                upd = score > bestv
                bestv = jnp.where(upd, score, bestv)
                besti = jnp.where(upd, jvec, besti)
                return bestv, besti

            bestv, besti = lax.fori_loop(
                0, NCAND // 16, cand_step,
                (jnp.full((16,), -jnp.inf, jnp.float32),
                 jnp.zeros((16,), jnp.int32)))

            m = jnp.max(bestv)
            cand = jnp.where(bestv == m, besti, jnp.int32(1 << 30))
            minj = jnp.min(cand)

            sel_idx = jnp.minimum(minj * 5 + 1 + iota, jnp.int32(ROW - 1))
            vals = plsc.load_gather(row_v, [sel_idx])
            plsc.store_scatter(o1v, [zeros + i * 4 + iota], vals, mask=iota < 4)
            sval = plsc.load_gather(
                srow_v, [jnp.minimum(zeros + (minj * 2 + 1),
                                     jnp.int32(SROW - 1))])
            plsc.store_scatter(o2v, [zeros + i], sval, mask=iota < 1)
            return carry

        lax.fori_loop(0, cnt, per_roi, jnp.int32(0))

        pltpu.sync_copy(o1v, out1_hbm.at[pl.ds(base * 4, RPW * 4)])
        pltpu.sync_copy(o2v, out2_hbm.at[pl.ds(base, RPW)])

    return trnms


_sc_trnms = _make_sc_call()


B = 40
PI = 3.141592653589793
NC = NCAND


def tc_body(rois_ref, rpn_ref, sc_ref, out1_ref, out2_ref):
    rp = rpn_ref[...]
    x1 = rp[1]
    y1 = rp[2]
    x2 = rp[3]
    y2 = rp[4]
    s1 = sc_ref[:, 1, :]
    r = rois_ref[...]
    rx1 = r[:, 0:1]
    ry1 = r[:, 1:2]
    rx2 = r[:, 2:3]
    ry2 = r[:, 3:4]

    x = (rx1 + rx2) / 2.0
    y = (ry1 + ry2) / 2.0
    w = jnp.abs(rx1 - rx2) + 0.0001
    h = jnp.abs(ry1 - ry2) + 0.0001
    p = (w + h) / 2.0
    s = jnp.sqrt((w + p) * (h + p))
    rr = w / h

    x_ = (x1 + x2) / 2.0
    y_ = (y1 + y2) / 2.0
    w_ = jnp.abs(x1 - x2) + 0.0001
    h_ = jnp.abs(y1 - y2) + 0.0001
    p_ = (w_ + h_) / 2.0
    s_ = jnp.sqrt((w_ + p_) * (h_ + p_))

    s_max = jnp.maximum(s / s_, s_ / s)
    r_max = jnp.maximum(rr, 1.0 / rr)
    penalty = jnp.exp(-K * (s_max * r_max - 1.0))
    ws = s * 2.0
    dist = jnp.sqrt((x - x_) ** 2 + (y - y_) ** 2)
    han = 0.5 + 0.5 * jnp.cos(dist * PI / ws)
    han = jnp.where(dist > ws, 0.0, han)
    pw = s1 * penalty + han * 0.42

    m = jnp.max(pw, axis=1, keepdims=True)
    li = lax.broadcasted_iota(jnp.int32, (B, NC), 1)
    cand = jnp.where(pw == m, li, jnp.int32(NC))
    jm = jnp.min(cand, axis=1, keepdims=True)
    onehot = li == jm

    def sel(v):
        return jnp.sum(jnp.where(onehot, v, 0.0), axis=1, keepdims=True)

    out1_ref[:, 0:1] = sel(x1)
    out1_ref[:, 1:2] = sel(y1)
    out1_ref[:, 2:3] = sel(x2)
    out1_ref[:, 3:4] = sel(y2)
    out2_ref[:, 0:1] = sel(s1)


def tc_kernel(rois, rpn_rois, scores):
    rpn_t = jnp.transpose(rpn_rois, (2, 0, 1))
    sc_t = jnp.transpose(scores, (0, 2, 1))
    grid = (N // B,)
    out1, out2 = pl.pallas_call(
        tc_body,
        grid=grid,
        in_specs=[
            pl.BlockSpec((B, 4), lambda i: (i, 0)),
            pl.BlockSpec((5, B, NC), lambda i: (0, i, 0)),
            pl.BlockSpec((B, 2, NC), lambda i: (i, 0, 0)),
        ],
        out_specs=[
            pl.BlockSpec((B, 4), lambda i: (i, 0)),
            pl.BlockSpec((B, 1), lambda i: (i, 0)),
        ],
        out_shape=[
            jax.ShapeDtypeStruct((N, 4), jnp.float32),
            jax.ShapeDtypeStruct((N, 1), jnp.float32),
        ],
    )(rois, rpn_t, sc_t)
    return out1, out2


@jax.jit
def kernel(rois, rpn_rois, scores):
    return tc_kernel(rois, rpn_rois, scores)


# speedup vs baseline: 85.4851x; 1.6130x over previous
import functools

import jax
import jax.numpy as jnp
from jax import lax
from jax.experimental import pallas as pl
from jax.experimental.pallas import tpu as pltpu
from jax.experimental.pallas import tpu_sc as plsc

N = 5000
NCAND = 2000
NW = 32
RPW = 160
NPAD = NW * RPW
ROW = NCAND * 5
SROW = NCAND * 2
NCHUNK = (RPW + 15) // 16

K = 0.055

_COS = (0.9999999999999956, -4.934802200543692, 4.058712126381724,
        -1.3352627683856102, 0.23533062721525902, -0.025806879372614583,
        0.0019295464417837132, -0.00010459814866408124, 4.268378688720986e-06)


def _rsqrt(x):
    i = lax.bitcast_convert_type(x, jnp.int32)
    i = jnp.int32(0x5F3759DF) - lax.shift_right_arithmetic(i, 1)
    y = lax.bitcast_convert_type(i, jnp.float32)
    for _ in range(3):
        y = y * (jnp.float32(1.5) - jnp.float32(0.5) * x * y * y)
    return y


def _coswin(v):
    acc = jnp.full_like(v, jnp.float32(_COS[-1]))
    for c in _COS[-2::-1]:
        acc = acc * v + jnp.float32(c)
    return acc


def _make_sc_call():
    mesh = plsc.VectorSubcoreMesh(core_axis_name="c", subcore_axis_name="s")

    @functools.partial(
        pl.kernel,
        mesh=mesh,
        compiler_params=pltpu.CompilerParams(needs_layout_passes=False),
        out_type=[
            jax.ShapeDtypeStruct((NPAD * 4,), jnp.float32),
            jax.ShapeDtypeStruct((NPAD,), jnp.float32),
        ],
        scratch_types=[
            pltpu.VMEM((RPW * 4,), jnp.float32),
            pltpu.VMEM((ROW,), jnp.float32),
            pltpu.VMEM((SROW,), jnp.float32),
            pltpu.VMEM((NCHUNK * 16,), jnp.float32),
            pltpu.VMEM((NCHUNK * 16,), jnp.float32),
            pltpu.VMEM((NCHUNK * 16,), jnp.float32),
            pltpu.VMEM((NCHUNK * 16,), jnp.float32),
            pltpu.VMEM((NCHUNK * 16,), jnp.float32),
            pltpu.VMEM((NCHUNK * 16,), jnp.float32),
            pltpu.VMEM((RPW * 4,), jnp.float32),
            pltpu.VMEM((RPW,), jnp.float32),
        ],
    )
    def trnms(rois_hbm, rpn_hbm, sc_hbm, out1_hbm, out2_hbm,
              rois_v, row_v, srow_v, cx, cy, cs, cis, cal, cga,
              o1v, o2v):
        wid = lax.axis_index("s") * 2 + lax.axis_index("c")
        base = wid * RPW
        cnt = jnp.minimum(jnp.int32(RPW), jnp.int32(N) - base)

        iota = lax.iota(jnp.int32, 16)
        iota5 = iota * 5
        iota2 = iota * 2
        zeros = jnp.zeros((16,), jnp.int32)

        pltpu.sync_copy(rois_hbm.at[pl.ds(base * 4, RPW * 4)], rois_v)

        for t in range(NCHUNK):
            li = jnp.minimum(iota + (16 * t), jnp.int32(RPW - 1)) * 4
            c0 = plsc.load_gather(rois_v, [li])
            c1 = plsc.load_gather(rois_v, [li + 1])
            c2 = plsc.load_gather(rois_v, [li + 2])
            c3 = plsc.load_gather(rois_v, [li + 3])
            x = (c0 + c2) * jnp.float32(0.5)
            y = (c1 + c3) * jnp.float32(0.5)
            w = jnp.abs(c0 - c2) + jnp.float32(1e-4)
            h = jnp.abs(c1 - c3) + jnp.float32(1e-4)
            p = (w + h) * jnp.float32(0.5)
            q = (w + p) * (h + p)
            rq = _rsqrt(q)
            s = q * rq
            r = w / h
            r_max = jnp.maximum(r, jnp.float32(1.0) / r)
            sl = pl.ds(16 * t, 16)
            cx[sl] = x
            cy[sl] = y
            cs[sl] = s
            cis[sl] = rq
            cal[sl] = jnp.float32(-K) * r_max
            cga[sl] = rq * rq * jnp.float32(0.25)

        def per_roi(i, carry):
            rid = base + i
            pltpu.sync_copy(rpn_hbm.at[pl.ds(rid * ROW, ROW)], row_v)
            pltpu.sync_copy(sc_hbm.at[pl.ds(rid * SROW, SROW)], srow_v)

            ii = zeros + i
            x = plsc.load_gather(cx, [ii])
            y = plsc.load_gather(cy, [ii])
            s = plsc.load_gather(cs, [ii])
            inv_s = plsc.load_gather(cis, [ii])
            alpha = plsc.load_gather(cal, [ii])
            gamma = plsc.load_gather(cga, [ii])

            def cand_step(jj, st):
                bestv, besti = st
                b80 = jj * 80
                c0 = plsc.load_gather(row_v, [iota5 + (b80 + 1)])
                c1 = plsc.load_gather(row_v, [iota5 + (b80 + 2)])
                c2 = plsc.load_gather(row_v, [iota5 + (b80 + 3)])
                c3 = plsc.load_gather(row_v, [iota5 + (b80 + 4)])
                s1 = plsc.load_gather(srow_v, [iota2 + (jj * 32 + 1)])

                x_ = (c0 + c2) * jnp.float32(0.5)
                y_ = (c1 + c3) * jnp.float32(0.5)
                w_ = jnp.abs(c0 - c2) + jnp.float32(1e-4)
                h_ = jnp.abs(c1 - c3) + jnp.float32(1e-4)
                p_ = (w_ + h_) * jnp.float32(0.5)
                a_ = (w_ + p_) * (h_ + p_)
                ra = _rsqrt(a_)
                s_max = jnp.maximum(s * ra, a_ * ra * inv_s)
                pen = jnp.exp(alpha * s_max + jnp.float32(K))

                dx = x - x_
                dy = y - y_
                v = (dx * dx + dy * dy) * gamma
                han42 = jnp.float32(0.21) * _coswin(v) + jnp.float32(0.21)
                han42 = jnp.where(v > jnp.float32(1.0), jnp.float32(0.0), han42)

                score = s1 * pen + han42
                jvec = iota + jj * 16
                upd = score > bestv
                bestv = jnp.where(upd, score, bestv)
                besti = jnp.where(upd, jvec, besti)
                return bestv, besti

            bestv, besti = lax.fori_loop(
                0, NCAND // 16, cand_step,
                (jnp.full((16,), -jnp.inf, jnp.float32),
                 jnp.zeros((16,), jnp.int32)))

            m = jnp.max(bestv)
            cand = jnp.where(bestv == m, besti, jnp.int32(1 << 30))
            minj = jnp.min(cand)

            sel_idx = jnp.minimum(minj * 5 + 1 + iota, jnp.int32(ROW - 1))
            vals = plsc.load_gather(row_v, [sel_idx])
            plsc.store_scatter(o1v, [zeros + i * 4 + iota], vals, mask=iota < 4)
            sval = plsc.load_gather(
                srow_v, [jnp.minimum(zeros + (minj * 2 + 1),
                                     jnp.int32(SROW - 1))])
            plsc.store_scatter(o2v, [zeros + i], sval, mask=iota < 1)
            return carry

        lax.fori_loop(0, cnt, per_roi, jnp.int32(0))

        pltpu.sync_copy(o1v, out1_hbm.at[pl.ds(base * 4, RPW * 4)])
        pltpu.sync_copy(o2v, out2_hbm.at[pl.ds(base, RPW)])

    return trnms


_sc_trnms = _make_sc_call()


B = 40
PI = 3.141592653589793
NC = NCAND


def tc_body(rois_ref, rp1_ref, rp2_ref, rp3_ref, rp4_ref, sc_ref,
            out1_ref, out2_ref):
    x1 = rp1_ref[0]
    y1 = rp2_ref[0]
    x2 = rp3_ref[0]
    y2 = rp4_ref[0]
    s1 = sc_ref[:, 1, :]
    r = rois_ref[...]
    rx1 = r[:, 0:1]
    ry1 = r[:, 1:2]
    rx2 = r[:, 2:3]
    ry2 = r[:, 3:4]

    x = (rx1 + rx2) / 2.0
    y = (ry1 + ry2) / 2.0
    w = jnp.abs(rx1 - rx2) + 0.0001
    h = jnp.abs(ry1 - ry2) + 0.0001
    p = (w + h) / 2.0
    s = jnp.sqrt((w + p) * (h + p))
    rr = w / h

    inv_s = 1.0 / s
    r_max = jnp.maximum(rr, 1.0 / rr)
    alpha = -K * r_max
    gamma = inv_s * inv_s * 0.25

    x_ = (x1 + x2) * 0.5
    y_ = (y1 + y2) * 0.5
    w_ = jnp.abs(x1 - x2) + 0.0001
    h_ = jnp.abs(y1 - y2) + 0.0001
    p_ = (w_ + h_) * 0.5
    a_ = (w_ + p_) * (h_ + p_)
    ra = lax.rsqrt(a_)
    s_max = jnp.maximum(s * ra, a_ * ra * inv_s)
    penalty = jnp.exp(alpha * s_max + K)

    dx = x - x_
    dy = y - y_
    v = (dx * dx + dy * dy) * gamma
    han42 = 0.21 * _coswin(v) + 0.21
    han42 = jnp.where(v > 1.0, 0.0, han42)
    pw = s1 * penalty + han42

    m = jnp.max(pw, axis=1, keepdims=True)
    li = lax.broadcasted_iota(jnp.int32, (B, NC), 1)
    cand = jnp.where(pw == m, li, jnp.int32(NC))
    jm = jnp.min(cand, axis=1, keepdims=True)
    onehot = li == jm

    def sel(v):
        return jnp.sum(jnp.where(onehot, v, 0.0), axis=1, keepdims=True)

    out1_ref[:, 0:1] = sel(x1)
    out1_ref[:, 1:2] = sel(y1)
    out1_ref[:, 2:3] = sel(x2)
    out1_ref[:, 3:4] = sel(y2)
    out2_ref[:, 0:1] = sel(s1)


def tc_kernel(rois, rpn_rois, scores):
    rpn_t = jnp.transpose(rpn_rois, (2, 0, 1))
    sc_t = jnp.transpose(scores, (0, 2, 1))
    grid = (N // B,)
    out1, out2 = pl.pallas_call(
        tc_body,
        grid=grid,
        in_specs=[
            pl.BlockSpec((B, 4), lambda i: (i, 0)),
            pl.BlockSpec((1, B, NC), lambda i: (1, i, 0)),
            pl.BlockSpec((1, B, NC), lambda i: (2, i, 0)),
            pl.BlockSpec((1, B, NC), lambda i: (3, i, 0)),
            pl.BlockSpec((1, B, NC), lambda i: (4, i, 0)),
            pl.BlockSpec((B, 2, NC), lambda i: (i, 0, 0)),
        ],
        out_specs=[
            pl.BlockSpec((B, 4), lambda i: (i, 0)),
            pl.BlockSpec((B, 1), lambda i: (i, 0)),
        ],
        out_shape=[
            jax.ShapeDtypeStruct((N, 4), jnp.float32),
            jax.ShapeDtypeStruct((N, 1), jnp.float32),
        ],
    )(rois, rpn_t, rpn_t, rpn_t, rpn_t, sc_t)
    return out1, out2


@jax.jit
def kernel(rois, rpn_rois, scores):
    return tc_kernel(rois, rpn_rois, scores)


# speedup vs baseline: 87.1117x; 1.0190x over previous
import functools

import jax
import jax.numpy as jnp
from jax import lax
from jax.experimental import pallas as pl
from jax.experimental.pallas import tpu as pltpu
from jax.experimental.pallas import tpu_sc as plsc

N = 5000
NCAND = 2000
NW = 32
NSC = 1024
RPW = NSC // NW
NTC0 = 1000

K = 0.055

_COS = (0.9999999999999956, -4.934802200543692, 4.058712126381724,
        -1.3352627683856102, 0.23533062721525902, -0.025806879372614583,
        0.0019295464417837132, -0.00010459814866408124, 4.268378688720986e-06)


def _rsqrt(x):
    i = lax.bitcast_convert_type(x, jnp.int32)
    i = jnp.int32(0x5F3759DF) - lax.shift_right_arithmetic(i, 1)
    y = lax.bitcast_convert_type(i, jnp.float32)
    for _ in range(3):
        y = y * (jnp.float32(1.5) - jnp.float32(0.5) * x * y * y)
    return y


def _coswin(v):
    acc = jnp.full_like(v, jnp.float32(_COS[-1]))
    for c in _COS[-2::-1]:
        acc = acc * v + jnp.float32(c)
    return acc


def _make_sc_call():
    mesh = plsc.VectorSubcoreMesh(core_axis_name="c", subcore_axis_name="s")

    @functools.partial(
        pl.kernel,
        mesh=mesh,
        compiler_params=pltpu.CompilerParams(
            needs_layout_passes=False, use_tc_tiling_on_sc=True),
        out_type=[
            jax.ShapeDtypeStruct((NSC * 4,), jnp.float32),
            jax.ShapeDtypeStruct((NSC,), jnp.float32),
        ],
        scratch_types=[
            pltpu.VMEM((RPW * 4,), jnp.float32),
            pltpu.VMEM((4, 8, NCAND), jnp.float32),
            pltpu.VMEM((8, NCAND), jnp.float32),
            pltpu.VMEM((RPW,), jnp.float32),
            pltpu.VMEM((RPW,), jnp.float32),
            pltpu.VMEM((RPW,), jnp.float32),
            pltpu.VMEM((RPW,), jnp.float32),
            pltpu.VMEM((RPW,), jnp.float32),
            pltpu.VMEM((RPW,), jnp.float32),
            pltpu.VMEM((RPW * 4,), jnp.float32),
            pltpu.VMEM((RPW,), jnp.float32),
        ],
    )
    def trnms(rois_hbm, rpn_hbm, s1_hbm, out1_hbm, out2_hbm,
              rois_v, pbuf, sbuf, cx, cy, cs, cis, cal, cga,
              o1v, o2v):
        wid = lax.axis_index("s") * 2 + lax.axis_index("c")
        base = wid * RPW

        iota = lax.iota(jnp.int32, 16)
        zeros = jnp.zeros((16,), jnp.int32)

        pltpu.sync_copy(rois_hbm.at[pl.ds(base * 4, RPW * 4)], rois_v)

        for t in range(RPW // 16):
            li = (iota + (16 * t)) * 4
            c0 = plsc.load_gather(rois_v, [li])
            c1 = plsc.load_gather(rois_v, [li + 1])
            c2 = plsc.load_gather(rois_v, [li + 2])
            c3 = plsc.load_gather(rois_v, [li + 3])
            x = (c0 + c2) * jnp.float32(0.5)
            y = (c1 + c3) * jnp.float32(0.5)
            w = jnp.abs(c0 - c2) + jnp.float32(1e-4)
            h = jnp.abs(c1 - c3) + jnp.float32(1e-4)
            p = (w + h) * jnp.float32(0.5)
            q = (w + p) * (h + p)
            rq = _rsqrt(q)
            s = q * rq
            r = w / h
            r_max = jnp.maximum(r, jnp.float32(1.0) / r)
            sl = pl.ds(16 * t, 16)
            cx[sl] = x
            cy[sl] = y
            cs[sl] = s
            cis[sl] = rq
            cal[sl] = jnp.float32(-K) * r_max
            cga[sl] = rq * rq * jnp.float32(0.25)

        for g in range(RPW // 8):
            r0 = base + g * 8
            pltpu.sync_copy(rpn_hbm.at[pl.ds(1, 4), pl.ds(r0, 8), :], pbuf)
            pltpu.sync_copy(s1_hbm.at[pl.ds(r0, 8), :], sbuf)

            def per_roi(r, carry):
                i = r + g * 8
                ii = zeros + i
                x = plsc.load_gather(cx, [ii])
                y = plsc.load_gather(cy, [ii])
                s = plsc.load_gather(cs, [ii])
                inv_s = plsc.load_gather(cis, [ii])
                alpha = plsc.load_gather(cal, [ii])
                gamma = plsc.load_gather(cga, [ii])

                def cand_step(jj, st):
                    bestv, besti = st
                    sl = pl.ds(jj * 16, 16)
                    c0 = pbuf[0, r, sl]
                    c1 = pbuf[1, r, sl]
                    c2 = pbuf[2, r, sl]
                    c3 = pbuf[3, r, sl]
                    s1 = sbuf[r, sl]

                    x_ = (c0 + c2) * jnp.float32(0.5)
                    y_ = (c1 + c3) * jnp.float32(0.5)
                    w_ = jnp.abs(c0 - c2) + jnp.float32(1e-4)
                    h_ = jnp.abs(c1 - c3) + jnp.float32(1e-4)
                    p_ = (w_ + h_) * jnp.float32(0.5)
                    a_ = (w_ + p_) * (h_ + p_)
                    ra = _rsqrt(a_)
                    s_max = jnp.maximum(s * ra, a_ * ra * inv_s)
                    pen = jnp.exp(alpha * s_max + jnp.float32(K))

                    dx = x - x_
                    dy = y - y_
                    v = (dx * dx + dy * dy) * gamma
                    han42 = jnp.float32(0.21) * _coswin(v) + jnp.float32(0.21)
                    han42 = jnp.where(
                        v > jnp.float32(1.0), jnp.float32(0.0), han42)

                    score = s1 * pen + han42
                    jvec = iota + jj * 16
                    upd = score > bestv
                    bestv = jnp.where(upd, score, bestv)
                    besti = jnp.where(upd, jvec, besti)
                    return bestv, besti

                bestv, besti = lax.fori_loop(
                    0, NCAND // 16, cand_step,
                    (jnp.full((16,), -jnp.inf, jnp.float32),
                     jnp.zeros((16,), jnp.int32)))

                m = jnp.max(bestv)
                cand = jnp.where(bestv == m, besti, jnp.int32(1 << 30))
                minj = jnp.min(cand)
                jc = jnp.minimum(minj, jnp.int32(NCAND - 16))
                lane = minj - jc

                for f in range(4):
                    vals = pbuf[f, r, pl.ds(jc, 16)]
                    plsc.store_scatter(
                        o1v, [zeros + (i * 4 + f)], vals, mask=iota == lane)
                sv = sbuf[r, pl.ds(jc, 16)]
                plsc.store_scatter(o2v, [zeros + i], sv, mask=iota == lane)
                return carry

            lax.fori_loop(0, 8, per_roi, jnp.int32(0))

        pltpu.sync_copy(o1v, out1_hbm.at[pl.ds(base * 4, RPW * 4)])
        pltpu.sync_copy(o2v, out2_hbm.at[pl.ds(base, RPW)])

    return trnms


_sc_trnms = _make_sc_call()


B = 40
PI = 3.141592653589793
NC = NCAND


def tc_body(rois_ref, rp1_ref, rp2_ref, rp3_ref, rp4_ref, sc_ref,
            out1_ref, out2_ref):
    x1 = rp1_ref[0]
    y1 = rp2_ref[0]
    x2 = rp3_ref[0]
    y2 = rp4_ref[0]
    s1 = sc_ref[:, 1, :]
    r = rois_ref[...]
    rx1 = r[:, 0:1]
    ry1 = r[:, 1:2]
    rx2 = r[:, 2:3]
    ry2 = r[:, 3:4]

    x = (rx1 + rx2) / 2.0
    y = (ry1 + ry2) / 2.0
    w = jnp.abs(rx1 - rx2) + 0.0001
    h = jnp.abs(ry1 - ry2) + 0.0001
    p = (w + h) / 2.0
    s = jnp.sqrt((w + p) * (h + p))
    rr = w / h

    inv_s = 1.0 / s
    r_max = jnp.maximum(rr, 1.0 / rr)
    alpha = -K * r_max
    gamma = inv_s * inv_s * 0.25

    x_ = (x1 + x2) * 0.5
    y_ = (y1 + y2) * 0.5
    w_ = jnp.abs(x1 - x2) + 0.0001
    h_ = jnp.abs(y1 - y2) + 0.0001
    p_ = (w_ + h_) * 0.5
    a_ = (w_ + p_) * (h_ + p_)
    ra = lax.rsqrt(a_)
    s_max = jnp.maximum(s * ra, a_ * ra * inv_s)
    penalty = jnp.exp(alpha * s_max + K)

    dx = x - x_
    dy = y - y_
    v = (dx * dx + dy * dy) * gamma
    han42 = 0.21 * _coswin(v) + 0.21
    han42 = jnp.where(v > 1.0, 0.0, han42)
    pw = s1 * penalty + han42

    m = jnp.max(pw, axis=1, keepdims=True)
    li = lax.broadcasted_iota(jnp.int32, (B, NC), 1)
    cand = jnp.where(pw == m, li, jnp.int32(NC))
    jm = jnp.min(cand, axis=1, keepdims=True)
    onehot = li == jm

    def sel(v):
        return jnp.sum(jnp.where(onehot, v, 0.0), axis=1, keepdims=True)

    out1_ref[:, 0:1] = sel(x1)
    out1_ref[:, 1:2] = sel(y1)
    out1_ref[:, 2:3] = sel(x2)
    out1_ref[:, 3:4] = sel(y2)
    out2_ref[:, 0:1] = sel(s1)


def tc_kernel(rois, rpn_t, sc_t, start, count):
    b0 = start // B
    grid = (count // B,)
    out1, out2 = pl.pallas_call(
        tc_body,
        grid=grid,
        in_specs=[
            pl.BlockSpec((B, 4), lambda i: (i + b0, 0)),
            pl.BlockSpec((1, B, NC), lambda i: (1, i + b0, 0)),
            pl.BlockSpec((1, B, NC), lambda i: (2, i + b0, 0)),
            pl.BlockSpec((1, B, NC), lambda i: (3, i + b0, 0)),
            pl.BlockSpec((1, B, NC), lambda i: (4, i + b0, 0)),
            pl.BlockSpec((B, 2, NC), lambda i: (i + b0, 0, 0)),
        ],
        out_specs=[
            pl.BlockSpec((B, 4), lambda i: (i, 0)),
            pl.BlockSpec((B, 1), lambda i: (i, 0)),
        ],
        out_shape=[
            jax.ShapeDtypeStruct((count, 4), jnp.float32),
            jax.ShapeDtypeStruct((count, 1), jnp.float32),
        ],
    )(rois, rpn_t, rpn_t, rpn_t, rpn_t, sc_t)
    return out1, out2


BE = 128


def _s1x_body(sc_ref, out_ref):
    out_ref[...] = sc_ref[:, 1, :]


def s1_extract(sc_t):
    return pl.pallas_call(
        _s1x_body,
        grid=(NSC // BE,),
        in_specs=[pl.BlockSpec((BE, 2, NC), lambda i: (i, 0, 0))],
        out_specs=pl.BlockSpec((BE, NC), lambda i: (i, 0)),
        out_shape=jax.ShapeDtypeStruct((NSC, NC), jnp.float32),
    )(sc_t)


@jax.jit
def kernel(rois, rpn_rois, scores):
    rpn_t = jnp.transpose(rpn_rois, (2, 0, 1))
    sc_t = jnp.transpose(scores, (0, 2, 1))
    s1x = s1_extract(sc_t)
    rois_sc = jnp.reshape(rois[:NSC], (-1,))
    o1sc, o2sc = _sc_trnms(rois_sc, rpn_t, s1x)
    t1, t2 = tc_kernel(rois, rpn_t, sc_t, NTC0, N - NTC0)
    out1 = jnp.concatenate([jnp.reshape(o1sc, (NSC, 4))[:NTC0], t1], axis=0)
    out2 = jnp.concatenate([o2sc[:NTC0, None], t2], axis=0)
    return out1, out2


# speedup vs baseline: 93.3522x; 1.0716x over previous
import functools

import jax
import jax.numpy as jnp
from jax import lax
from jax.experimental import pallas as pl
from jax.experimental.pallas import tpu as pltpu
from jax.experimental.pallas import tpu_sc as plsc

N = 5000
NCAND = 2000
NW = 32
NSC = 1536
RPW = NSC // NW
NTC0 = 1520

K = 0.055

_COS = (0.9999999999999956, -4.934802200543692, 4.058712126381724,
        -1.3352627683856102, 0.23533062721525902, -0.025806879372614583,
        0.0019295464417837132, -0.00010459814866408124, 4.268378688720986e-06)


def _rsqrt(x):
    i = lax.bitcast_convert_type(x, jnp.int32)
    i = jnp.int32(0x5F3759DF) - lax.shift_right_arithmetic(i, 1)
    y = lax.bitcast_convert_type(i, jnp.float32)
    for _ in range(3):
        y = y * (jnp.float32(1.5) - jnp.float32(0.5) * x * y * y)
    return y


def _coswin(v):
    acc = jnp.full_like(v, jnp.float32(_COS[-1]))
    for c in _COS[-2::-1]:
        acc = acc * v + jnp.float32(c)
    return acc


def _make_sc_call():
    mesh = plsc.VectorSubcoreMesh(core_axis_name="c", subcore_axis_name="s")

    @functools.partial(
        pl.kernel,
        mesh=mesh,
        compiler_params=pltpu.CompilerParams(
            needs_layout_passes=False, use_tc_tiling_on_sc=True),
        out_type=[
            jax.ShapeDtypeStruct((NSC * 4,), jnp.float32),
            jax.ShapeDtypeStruct((NSC,), jnp.float32),
        ],
        scratch_types=[
            pltpu.VMEM((RPW * 4,), jnp.float32),
            pltpu.VMEM((4, 8, NCAND), jnp.float32),
            pltpu.VMEM((8, NCAND), jnp.float32),
            pltpu.VMEM((RPW,), jnp.float32),
            pltpu.VMEM((RPW,), jnp.float32),
            pltpu.VMEM((RPW,), jnp.float32),
            pltpu.VMEM((RPW,), jnp.float32),
            pltpu.VMEM((RPW,), jnp.float32),
            pltpu.VMEM((RPW,), jnp.float32),
            pltpu.VMEM((RPW * 4,), jnp.float32),
            pltpu.VMEM((RPW,), jnp.float32),
        ],
    )
    def trnms(rois_hbm, rpn_hbm, s1_hbm, out1_hbm, out2_hbm,
              rois_v, pbuf, sbuf, cx, cy, cs, cis, cal, cga,
              o1v, o2v):
        wid = lax.axis_index("s") * 2 + lax.axis_index("c")
        base = wid * RPW

        iota = lax.iota(jnp.int32, 16)
        zeros = jnp.zeros((16,), jnp.int32)

        pltpu.sync_copy(rois_hbm.at[pl.ds(base * 4, RPW * 4)], rois_v)

        for t in range(RPW // 16):
            li = (iota + (16 * t)) * 4
            c0 = plsc.load_gather(rois_v, [li])
            c1 = plsc.load_gather(rois_v, [li + 1])
            c2 = plsc.load_gather(rois_v, [li + 2])
            c3 = plsc.load_gather(rois_v, [li + 3])
            x = (c0 + c2) * jnp.float32(0.5)
            y = (c1 + c3) * jnp.float32(0.5)
            w = jnp.abs(c0 - c2) + jnp.float32(1e-4)
            h = jnp.abs(c1 - c3) + jnp.float32(1e-4)
            p = (w + h) * jnp.float32(0.5)
            q = (w + p) * (h + p)
            rq = _rsqrt(q)
            s = q * rq
            r = w / h
            r_max = jnp.maximum(r, jnp.float32(1.0) / r)
            sl = pl.ds(16 * t, 16)
            cx[sl] = x
            cy[sl] = y
            cs[sl] = s
            cis[sl] = rq
            cal[sl] = jnp.float32(-K) * r_max
            cga[sl] = rq * rq * jnp.float32(0.25)

        for g in range(RPW // 8):
            r0 = base + g * 8
            pltpu.sync_copy(rpn_hbm.at[pl.ds(1, 4), pl.ds(r0, 8), :], pbuf)
            pltpu.sync_copy(s1_hbm.at[pl.ds(r0, 8), :], sbuf)

            def per_roi(r, carry):
                i = r + g * 8
                ii = zeros + i
                x = plsc.load_gather(cx, [ii])
                y = plsc.load_gather(cy, [ii])
                s = plsc.load_gather(cs, [ii])
                inv_s = plsc.load_gather(cis, [ii])
                alpha = plsc.load_gather(cal, [ii])
                gamma = plsc.load_gather(cga, [ii])

                def cand_step(jj, st):
                    bestv, besti = st
                    sl = pl.ds(jj * 16, 16)
                    c0 = pbuf[0, r, sl]
                    c1 = pbuf[1, r, sl]
                    c2 = pbuf[2, r, sl]
                    c3 = pbuf[3, r, sl]
                    s1 = sbuf[r, sl]

                    x_ = (c0 + c2) * jnp.float32(0.5)
                    y_ = (c1 + c3) * jnp.float32(0.5)
                    w_ = jnp.abs(c0 - c2) + jnp.float32(1e-4)
                    h_ = jnp.abs(c1 - c3) + jnp.float32(1e-4)
                    p_ = (w_ + h_) * jnp.float32(0.5)
                    a_ = (w_ + p_) * (h_ + p_)
                    ra = _rsqrt(a_)
                    s_max = jnp.maximum(s * ra, a_ * ra * inv_s)
                    pen = jnp.exp(alpha * s_max + jnp.float32(K))

                    dx = x - x_
                    dy = y - y_
                    v = (dx * dx + dy * dy) * gamma
                    han42 = jnp.float32(0.21) * _coswin(v) + jnp.float32(0.21)
                    han42 = jnp.where(
                        v > jnp.float32(1.0), jnp.float32(0.0), han42)

                    score = s1 * pen + han42
                    jvec = iota + jj * 16
                    upd = score > bestv
                    bestv = jnp.where(upd, score, bestv)
                    besti = jnp.where(upd, jvec, besti)
                    return bestv, besti

                bestv, besti = lax.fori_loop(
                    0, NCAND // 16, cand_step,
                    (jnp.full((16,), -jnp.inf, jnp.float32),
                     jnp.zeros((16,), jnp.int32)))

                m = jnp.max(bestv)
                cand = jnp.where(bestv == m, besti, jnp.int32(1 << 30))
                minj = jnp.min(cand)
                jc = jnp.minimum(minj, jnp.int32(NCAND - 16))
                lane = minj - jc

                for f in range(4):
                    vals = pbuf[f, r, pl.ds(jc, 16)]
                    plsc.store_scatter(
                        o1v, [zeros + (i * 4 + f)], vals, mask=iota == lane)
                sv = sbuf[r, pl.ds(jc, 16)]
                plsc.store_scatter(o2v, [zeros + i], sv, mask=iota == lane)
                return carry

            lax.fori_loop(0, 8, per_roi, jnp.int32(0))

        pltpu.sync_copy(o1v, out1_hbm.at[pl.ds(base * 4, RPW * 4)])
        pltpu.sync_copy(o2v, out2_hbm.at[pl.ds(base, RPW)])

    return trnms


_sc_trnms = _make_sc_call()


B = 40
PI = 3.141592653589793
NC = NCAND


def tc_body(rois_ref, rp1_ref, rp2_ref, rp3_ref, rp4_ref, sc_ref,
            out1_ref, out2_ref):
    x1 = rp1_ref[0]
    y1 = rp2_ref[0]
    x2 = rp3_ref[0]
    y2 = rp4_ref[0]
    s1 = sc_ref[:, 1, :]
    r = rois_ref[...]
    rx1 = r[:, 0:1]
    ry1 = r[:, 1:2]
    rx2 = r[:, 2:3]
    ry2 = r[:, 3:4]

    x = (rx1 + rx2) / 2.0
    y = (ry1 + ry2) / 2.0
    w = jnp.abs(rx1 - rx2) + 0.0001
    h = jnp.abs(ry1 - ry2) + 0.0001
    p = (w + h) / 2.0
    s = jnp.sqrt((w + p) * (h + p))
    rr = w / h

    inv_s = 1.0 / s
    r_max = jnp.maximum(rr, 1.0 / rr)
    alpha = -K * r_max
    gamma = inv_s * inv_s * 0.25

    x_ = (x1 + x2) * 0.5
    y_ = (y1 + y2) * 0.5
    w_ = jnp.abs(x1 - x2) + 0.0001
    h_ = jnp.abs(y1 - y2) + 0.0001
    p_ = (w_ + h_) * 0.5
    a_ = (w_ + p_) * (h_ + p_)
    ra = lax.rsqrt(a_)
    s_max = jnp.maximum(s * ra, a_ * ra * inv_s)
    penalty = jnp.exp(alpha * s_max + K)

    dx = x - x_
    dy = y - y_
    v = (dx * dx + dy * dy) * gamma
    han42 = 0.21 * _coswin(v) + 0.21
    han42 = jnp.where(v > 1.0, 0.0, han42)
    pw = s1 * penalty + han42

    m = jnp.max(pw, axis=1, keepdims=True)
    li = lax.broadcasted_iota(jnp.int32, (B, NC), 1)
    cand = jnp.where(pw == m, li, jnp.int32(NC))
    jm = jnp.min(cand, axis=1, keepdims=True)
    onehot = li == jm

    def sel(v):
        return jnp.sum(jnp.where(onehot, v, 0.0), axis=1, keepdims=True)

    out1_ref[:, 0:1] = sel(x1)
    out1_ref[:, 1:2] = sel(y1)
    out1_ref[:, 2:3] = sel(x2)
    out1_ref[:, 3:4] = sel(y2)
    out2_ref[:, 0:1] = sel(s1)


def tc_kernel(rois, rpn_t, sc_t, start, count):
    b0 = start // B
    grid = (count // B,)
    out1, out2 = pl.pallas_call(
        tc_body,
        grid=grid,
        in_specs=[
            pl.BlockSpec((B, 4), lambda i: (i + b0, 0)),
            pl.BlockSpec((1, B, NC), lambda i: (1, i + b0, 0)),
            pl.BlockSpec((1, B, NC), lambda i: (2, i + b0, 0)),
            pl.BlockSpec((1, B, NC), lambda i: (3, i + b0, 0)),
            pl.BlockSpec((1, B, NC), lambda i: (4, i + b0, 0)),
            pl.BlockSpec((B, 2, NC), lambda i: (i + b0, 0, 0)),
        ],
        out_specs=[
            pl.BlockSpec((B, 4), lambda i: (i, 0)),
            pl.BlockSpec((B, 1), lambda i: (i, 0)),
        ],
        out_shape=[
            jax.ShapeDtypeStruct((count, 4), jnp.float32),
            jax.ShapeDtypeStruct((count, 1), jnp.float32),
        ],
    )(rois, rpn_t, rpn_t, rpn_t, rpn_t, sc_t)
    return out1, out2


BE = 128


def _s1x_body(sc_ref, out_ref):
    out_ref[...] = sc_ref[:, 1, :]


def s1_extract(sc_t):
    return pl.pallas_call(
        _s1x_body,
        grid=(NSC // BE,),
        in_specs=[pl.BlockSpec((BE, 2, NC), lambda i: (i, 0, 0))],
        out_specs=pl.BlockSpec((BE, NC), lambda i: (i, 0)),
        out_shape=jax.ShapeDtypeStruct((NSC, NC), jnp.float32),
    )(sc_t)


@jax.jit
def kernel(rois, rpn_rois, scores):
    rpn_t = jnp.transpose(rpn_rois, (2, 0, 1))
    sc_t = jnp.transpose(scores, (0, 2, 1))
    s1x = s1_extract(sc_t)
    rois_sc = jnp.reshape(rois[:NSC], (-1,))
    o1sc, o2sc = _sc_trnms(rois_sc, rpn_t, s1x)
    t1, t2 = tc_kernel(rois, rpn_t, sc_t, NTC0, N - NTC0)
    out1 = jnp.concatenate([jnp.reshape(o1sc, (NSC, 4))[:NTC0], t1], axis=0)
    out2 = jnp.concatenate([o2sc[:NTC0, None], t2], axis=0)
    return out1, out2


# speedup vs baseline: 101.8716x; 1.0913x over previous
import functools

import jax
import jax.numpy as jnp
from jax import lax
from jax.experimental import pallas as pl
from jax.experimental.pallas import tpu as pltpu
from jax.experimental.pallas import tpu_sc as plsc

N = 5000
NCAND = 2000
NW = 32
NSC = 1536
RPW = NSC // NW
NTC0 = 1520

K = 0.055

_COS = (0.9999999999999956, -4.934802200543692, 4.058712126381724,
        -1.3352627683856102, 0.23533062721525902, -0.025806879372614583,
        0.0019295464417837132, -0.00010459814866408124, 4.268378688720986e-06)


def _rsqrt(x):
    i = lax.bitcast_convert_type(x, jnp.int32)
    i = jnp.int32(0x5F3759DF) - lax.shift_right_arithmetic(i, 1)
    y = lax.bitcast_convert_type(i, jnp.float32)
    for _ in range(3):
        y = y * (jnp.float32(1.5) - jnp.float32(0.5) * x * y * y)
    return y


def _coswin(v):
    acc = jnp.full_like(v, jnp.float32(_COS[-1]))
    for c in _COS[-2::-1]:
        acc = acc * v + jnp.float32(c)
    return acc


def _make_sc_call():
    mesh = plsc.VectorSubcoreMesh(core_axis_name="c", subcore_axis_name="s")

    @functools.partial(
        pl.kernel,
        mesh=mesh,
        compiler_params=pltpu.CompilerParams(
            needs_layout_passes=False, use_tc_tiling_on_sc=True),
        out_type=[
            jax.ShapeDtypeStruct((NSC * 4,), jnp.float32),
            jax.ShapeDtypeStruct((NSC,), jnp.float32),
        ],
        scratch_types=[
            pltpu.VMEM((RPW * 4,), jnp.float32),
            pltpu.VMEM((4, 8, NCAND), jnp.float32),
            pltpu.VMEM((8, 2, NCAND), jnp.float32),
            pltpu.VMEM((RPW,), jnp.float32),
            pltpu.VMEM((RPW,), jnp.float32),
            pltpu.VMEM((RPW,), jnp.float32),
            pltpu.VMEM((RPW,), jnp.float32),
            pltpu.VMEM((RPW,), jnp.float32),
            pltpu.VMEM((RPW,), jnp.float32),
            pltpu.VMEM((RPW * 4,), jnp.float32),
            pltpu.VMEM((RPW,), jnp.float32),
        ],
    )
    def trnms(rois_hbm, rpn_hbm, s1_hbm, out1_hbm, out2_hbm,
              rois_v, pbuf, sbuf, cx, cy, cs, cis, cal, cga,
              o1v, o2v):
        wid = lax.axis_index("s") * 2 + lax.axis_index("c")
        base = wid * RPW

        iota = lax.iota(jnp.int32, 16)
        zeros = jnp.zeros((16,), jnp.int32)

        pltpu.sync_copy(rois_hbm.at[pl.ds(base * 4, RPW * 4)], rois_v)

        for t in range(RPW // 16):
            li = (iota + (16 * t)) * 4
            c0 = plsc.load_gather(rois_v, [li])
            c1 = plsc.load_gather(rois_v, [li + 1])
            c2 = plsc.load_gather(rois_v, [li + 2])
            c3 = plsc.load_gather(rois_v, [li + 3])
            x = (c0 + c2) * jnp.float32(0.5)
            y = (c1 + c3) * jnp.float32(0.5)
            w = jnp.abs(c0 - c2) + jnp.float32(1e-4)
            h = jnp.abs(c1 - c3) + jnp.float32(1e-4)
            p = (w + h) * jnp.float32(0.5)
            q = (w + p) * (h + p)
            rq = _rsqrt(q)
            s = q * rq
            r = w / h
            r_max = jnp.maximum(r, jnp.float32(1.0) / r)
            sl = pl.ds(16 * t, 16)
            cx[sl] = x
            cy[sl] = y
            cs[sl] = s
            cis[sl] = rq
            cal[sl] = jnp.float32(-K) * r_max
            cga[sl] = rq * rq * jnp.float32(0.25)

        for g in range(RPW // 8):
            r0 = base + g * 8
            pltpu.sync_copy(rpn_hbm.at[pl.ds(1, 4), pl.ds(r0, 8), :], pbuf)
            pltpu.sync_copy(s1_hbm.at[pl.ds(r0, 8), :, :], sbuf)

            def per_roi(r, carry):
                i = r + g * 8
                ii = zeros + i
                x = plsc.load_gather(cx, [ii])
                y = plsc.load_gather(cy, [ii])
                s = plsc.load_gather(cs, [ii])
                inv_s = plsc.load_gather(cis, [ii])
                alpha = plsc.load_gather(cal, [ii])
                gamma = plsc.load_gather(cga, [ii])

                def cand_step(jj, st):
                    bestv, besti = st
                    sl = pl.ds(jj * 16, 16)
                    c0 = pbuf[0, r, sl]
                    c1 = pbuf[1, r, sl]
                    c2 = pbuf[2, r, sl]
                    c3 = pbuf[3, r, sl]
                    s1 = sbuf[r, 1, sl]

                    x_ = (c0 + c2) * jnp.float32(0.5)
                    y_ = (c1 + c3) * jnp.float32(0.5)
                    w_ = jnp.abs(c0 - c2) + jnp.float32(1e-4)
                    h_ = jnp.abs(c1 - c3) + jnp.float32(1e-4)
                    p_ = (w_ + h_) * jnp.float32(0.5)
                    a_ = (w_ + p_) * (h_ + p_)
                    ra = _rsqrt(a_)
                    s_max = jnp.maximum(s * ra, a_ * ra * inv_s)
                    pen = jnp.exp(alpha * s_max + jnp.float32(K))

                    dx = x - x_
                    dy = y - y_
                    v = (dx * dx + dy * dy) * gamma
                    han42 = jnp.float32(0.21) * _coswin(v) + jnp.float32(0.21)
                    han42 = jnp.where(
                        v > jnp.float32(1.0), jnp.float32(0.0), han42)

                    score = s1 * pen + han42
                    jvec = iota + jj * 16
                    upd = score > bestv
                    bestv = jnp.where(upd, score, bestv)
                    besti = jnp.where(upd, jvec, besti)
                    return bestv, besti

                bestv, besti = lax.fori_loop(
                    0, NCAND // 16, cand_step,
                    (jnp.full((16,), -jnp.inf, jnp.float32),
                     jnp.zeros((16,), jnp.int32)))

                m = jnp.max(bestv)
                cand = jnp.where(bestv == m, besti, jnp.int32(1 << 30))
                minj = jnp.min(cand)
                jc = jnp.minimum(minj, jnp.int32(NCAND - 16))
                lane = minj - jc

                for f in range(4):
                    vals = pbuf[f, r, pl.ds(jc, 16)]
                    plsc.store_scatter(
                        o1v, [zeros + (i * 4 + f)], vals, mask=iota == lane)
                sv = sbuf[r, 1, pl.ds(jc, 16)]
                plsc.store_scatter(o2v, [zeros + i], sv, mask=iota == lane)
                return carry

            lax.fori_loop(0, 8, per_roi, jnp.int32(0))

        pltpu.sync_copy(o1v, out1_hbm.at[pl.ds(base * 4, RPW * 4)])
        pltpu.sync_copy(o2v, out2_hbm.at[pl.ds(base, RPW)])

    return trnms


_sc_trnms = _make_sc_call()


B = 40
PI = 3.141592653589793
NC = NCAND


def tc_body(rois_ref, rp1_ref, rp2_ref, rp3_ref, rp4_ref, sc_ref,
            out1_ref, out2_ref):
    x1 = rp1_ref[0]
    y1 = rp2_ref[0]
    x2 = rp3_ref[0]
    y2 = rp4_ref[0]
    s1 = sc_ref[:, 1, :]
    r = rois_ref[...]
    rx1 = r[:, 0:1]
    ry1 = r[:, 1:2]
    rx2 = r[:, 2:3]
    ry2 = r[:, 3:4]

    x = (rx1 + rx2) / 2.0
    y = (ry1 + ry2) / 2.0
    w = jnp.abs(rx1 - rx2) + 0.0001
    h = jnp.abs(ry1 - ry2) + 0.0001
    p = (w + h) / 2.0
    s = jnp.sqrt((w + p) * (h + p))
    rr = w / h

    inv_s = 1.0 / s
    r_max = jnp.maximum(rr, 1.0 / rr)
    alpha = -K * r_max
    gamma = inv_s * inv_s * 0.25

    x_ = (x1 + x2) * 0.5
    y_ = (y1 + y2) * 0.5
    w_ = jnp.abs(x1 - x2) + 0.0001
    h_ = jnp.abs(y1 - y2) + 0.0001
    p_ = (w_ + h_) * 0.5
    a_ = (w_ + p_) * (h_ + p_)
    ra = lax.rsqrt(a_)
    s_max = jnp.maximum(s * ra, a_ * ra * inv_s)
    penalty = jnp.exp(alpha * s_max + K)

    dx = x - x_
    dy = y - y_
    v = (dx * dx + dy * dy) * gamma
    han42 = 0.21 * _coswin(v) + 0.21
    han42 = jnp.where(v > 1.0, 0.0, han42)
    pw = s1 * penalty + han42

    m = jnp.max(pw, axis=1, keepdims=True)
    li = lax.broadcasted_iota(jnp.int32, (B, NC), 1)
    cand = jnp.where(pw == m, li, jnp.int32(NC))
    jm = jnp.min(cand, axis=1, keepdims=True)
    onehot = li == jm

    def sel(v):
        return jnp.sum(jnp.where(onehot, v, 0.0), axis=1, keepdims=True)

    out1_ref[:, 0:1] = sel(x1)
    out1_ref[:, 1:2] = sel(y1)
    out1_ref[:, 2:3] = sel(x2)
    out1_ref[:, 3:4] = sel(y2)
    out2_ref[:, 0:1] = sel(s1)


def tc_kernel(rois, rpn_t, sc_t, start, count):
    b0 = start // B
    grid = (count // B,)
    out1, out2 = pl.pallas_call(
        tc_body,
        grid=grid,
        in_specs=[
            pl.BlockSpec((B, 4), lambda i: (i + b0, 0)),
            pl.BlockSpec((1, B, NC), lambda i: (1, i + b0, 0)),
            pl.BlockSpec((1, B, NC), lambda i: (2, i + b0, 0)),
            pl.BlockSpec((1, B, NC), lambda i: (3, i + b0, 0)),
            pl.BlockSpec((1, B, NC), lambda i: (4, i + b0, 0)),
            pl.BlockSpec((B, 2, NC), lambda i: (i + b0, 0, 0)),
        ],
        out_specs=[
            pl.BlockSpec((B, 4), lambda i: (i, 0)),
            pl.BlockSpec((B, 1), lambda i: (i, 0)),
        ],
        out_shape=[
            jax.ShapeDtypeStruct((count, 4), jnp.float32),
            jax.ShapeDtypeStruct((count, 1), jnp.float32),
        ],
    )(rois, rpn_t, rpn_t, rpn_t, rpn_t, sc_t)
    return out1, out2


BE = 128


def _s1x_body(sc_ref, out_ref):
    out_ref[...] = sc_ref[:, 1, :]


def s1_extract(sc_t):
    return pl.pallas_call(
        _s1x_body,
        grid=(NSC // BE,),
        in_specs=[pl.BlockSpec((BE, 2, NC), lambda i: (i, 0, 0))],
        out_specs=pl.BlockSpec((BE, NC), lambda i: (i, 0)),
        out_shape=jax.ShapeDtypeStruct((NSC, NC), jnp.float32),
    )(sc_t)


@jax.jit
def kernel(rois, rpn_rois, scores):
    rpn_t = jnp.transpose(rpn_rois, (2, 0, 1))
    sc_t = jnp.transpose(scores, (0, 2, 1))
    rois_sc = jnp.reshape(rois[:NSC], (-1,))
    o1sc, o2sc = _sc_trnms(rois_sc, rpn_t, sc_t)
    t1, t2 = tc_kernel(rois, rpn_t, sc_t, NTC0, N - NTC0)
    out1 = jnp.concatenate([jnp.reshape(o1sc, (NSC, 4))[:NTC0], t1], axis=0)
    out2 = jnp.concatenate([o2sc[:NTC0, None], t2], axis=0)
    return out1, out2


# speedup vs baseline: 103.3824x; 1.0148x over previous
import functools

import jax
import jax.numpy as jnp
from jax import lax
from jax.experimental import pallas as pl
from jax.experimental.pallas import tpu as pltpu
from jax.experimental.pallas import tpu_sc as plsc

N = 5000
NCAND = 2000
NW = 32
NSC = 1536
RPW = NSC // NW
NTC0 = 1520

K = 0.055

_COS = (0.9999999999999956, -4.934802200543692, 4.058712126381724,
        -1.3352627683856102, 0.23533062721525902, -0.025806879372614583,
        0.0019295464417837132, -0.00010459814866408124, 4.268378688720986e-06)


def _rsqrt(x):
    i = lax.bitcast_convert_type(x, jnp.int32)
    i = jnp.int32(0x5F3759DF) - lax.shift_right_arithmetic(i, 1)
    y = lax.bitcast_convert_type(i, jnp.float32)
    for _ in range(3):
        y = y * (jnp.float32(1.5) - jnp.float32(0.5) * x * y * y)
    return y


def _coswin(v):
    acc = jnp.full_like(v, jnp.float32(_COS[-1]))
    for c in _COS[-2::-1]:
        acc = acc * v + jnp.float32(c)
    return acc


def _make_sc_call():
    mesh = plsc.VectorSubcoreMesh(core_axis_name="c", subcore_axis_name="s")

    @functools.partial(
        pl.kernel,
        mesh=mesh,
        compiler_params=pltpu.CompilerParams(
            needs_layout_passes=False, use_tc_tiling_on_sc=True),
        out_type=[
            jax.ShapeDtypeStruct((NSC * 4,), jnp.float32),
            jax.ShapeDtypeStruct((NSC,), jnp.float32),
        ],
        scratch_types=[
            pltpu.VMEM((RPW, 4), jnp.float32),
            pltpu.VMEM((4, 8, NCAND), jnp.float32),
            pltpu.VMEM((8, 2, NCAND), jnp.float32),
            pltpu.VMEM((RPW,), jnp.float32),
            pltpu.VMEM((RPW,), jnp.float32),
            pltpu.VMEM((RPW,), jnp.float32),
            pltpu.VMEM((RPW,), jnp.float32),
            pltpu.VMEM((RPW,), jnp.float32),
            pltpu.VMEM((RPW,), jnp.float32),
            pltpu.VMEM((RPW * 4,), jnp.float32),
            pltpu.VMEM((RPW,), jnp.float32),
        ],
    )
    def trnms(rois_hbm, rpn_hbm, s1_hbm, out1_hbm, out2_hbm,
              rois_v, pbuf, sbuf, cx, cy, cs, cis, cal, cga,
              o1v, o2v):
        wid = lax.axis_index("s") * 2 + lax.axis_index("c")
        base = wid * RPW

        iota = lax.iota(jnp.int32, 16)
        zeros = jnp.zeros((16,), jnp.int32)

        pltpu.sync_copy(rois_hbm.at[pl.ds(base, RPW), :], rois_v)

        for t in range(RPW // 16):
            li = iota + (16 * t)
            c0 = plsc.load_gather(rois_v, [li, zeros])
            c1 = plsc.load_gather(rois_v, [li, zeros + 1])
            c2 = plsc.load_gather(rois_v, [li, zeros + 2])
            c3 = plsc.load_gather(rois_v, [li, zeros + 3])
            x = (c0 + c2) * jnp.float32(0.5)
            y = (c1 + c3) * jnp.float32(0.5)
            w = jnp.abs(c0 - c2) + jnp.float32(1e-4)
            h = jnp.abs(c1 - c3) + jnp.float32(1e-4)
            p = (w + h) * jnp.float32(0.5)
            q = (w + p) * (h + p)
            rq = _rsqrt(q)
            s = q * rq
            r = w / h
            r_max = jnp.maximum(r, jnp.float32(1.0) / r)
            sl = pl.ds(16 * t, 16)
            cx[sl] = x
            cy[sl] = y
            cs[sl] = s
            cis[sl] = rq
            cal[sl] = jnp.float32(-K) * r_max
            cga[sl] = rq * rq * jnp.float32(0.25)

        for g in range(RPW // 8):
            r0 = base + g * 8
            pltpu.sync_copy(rpn_hbm.at[pl.ds(1, 4), pl.ds(r0, 8), :], pbuf)
            pltpu.sync_copy(s1_hbm.at[pl.ds(r0, 8), :, :], sbuf)

            def per_roi(r, carry):
                i = r + g * 8
                ii = zeros + i
                x = plsc.load_gather(cx, [ii])
                y = plsc.load_gather(cy, [ii])
                s = plsc.load_gather(cs, [ii])
                inv_s = plsc.load_gather(cis, [ii])
                alpha = plsc.load_gather(cal, [ii])
                gamma = plsc.load_gather(cga, [ii])

                def cand_step(jj, st):
                    bestv, besti = st
                    sl = pl.ds(jj * 16, 16)
                    c0 = pbuf[0, r, sl]
                    c1 = pbuf[1, r, sl]
                    c2 = pbuf[2, r, sl]
                    c3 = pbuf[3, r, sl]
                    s1 = sbuf[r, 1, sl]

                    x_ = (c0 + c2) * jnp.float32(0.5)
                    y_ = (c1 + c3) * jnp.float32(0.5)
                    w_ = jnp.abs(c0 - c2) + jnp.float32(1e-4)
                    h_ = jnp.abs(c1 - c3) + jnp.float32(1e-4)
                    p_ = (w_ + h_) * jnp.float32(0.5)
                    a_ = (w_ + p_) * (h_ + p_)
                    ra = _rsqrt(a_)
                    s_max = jnp.maximum(s * ra, a_ * ra * inv_s)
                    pen = jnp.exp(alpha * s_max + jnp.float32(K))

                    dx = x - x_
                    dy = y - y_
                    v = (dx * dx + dy * dy) * gamma
                    han42 = jnp.float32(0.21) * _coswin(v) + jnp.float32(0.21)
                    han42 = jnp.where(
                        v > jnp.float32(1.0), jnp.float32(0.0), han42)

                    score = s1 * pen + han42
                    jvec = iota + jj * 16
                    upd = score > bestv
                    bestv = jnp.where(upd, score, bestv)
                    besti = jnp.where(upd, jvec, besti)
                    return bestv, besti

                bestv, besti = lax.fori_loop(
                    0, NCAND // 16, cand_step,
                    (jnp.full((16,), -jnp.inf, jnp.float32),
                     jnp.zeros((16,), jnp.int32)))

                m = jnp.max(bestv)
                cand = jnp.where(bestv == m, besti, jnp.int32(1 << 30))
                minj = jnp.min(cand)
                jc = jnp.minimum(minj, jnp.int32(NCAND - 16))
                lane = minj - jc

                for f in range(4):
                    vals = pbuf[f, r, pl.ds(jc, 16)]
                    plsc.store_scatter(
                        o1v, [zeros + (i * 4 + f)], vals, mask=iota == lane)
                sv = sbuf[r, 1, pl.ds(jc, 16)]
                plsc.store_scatter(o2v, [zeros + i], sv, mask=iota == lane)
                return carry

            lax.fori_loop(0, 8, per_roi, jnp.int32(0))

        pltpu.sync_copy(o1v, out1_hbm.at[pl.ds(base * 4, RPW * 4)])
        pltpu.sync_copy(o2v, out2_hbm.at[pl.ds(base, RPW)])

    return trnms


_sc_trnms = _make_sc_call()


B = 40
PI = 3.141592653589793
NC = NCAND


def tc_body(rois_ref, rp1_ref, rp2_ref, rp3_ref, rp4_ref, sc_ref,
            out1_ref, out2_ref):
    x1 = rp1_ref[0]
    y1 = rp2_ref[0]
    x2 = rp3_ref[0]
    y2 = rp4_ref[0]
    s1 = sc_ref[:, 1, :]
    r = rois_ref[...]
    rx1 = r[:, 0:1]
    ry1 = r[:, 1:2]
    rx2 = r[:, 2:3]
    ry2 = r[:, 3:4]

    x = (rx1 + rx2) / 2.0
    y = (ry1 + ry2) / 2.0
    w = jnp.abs(rx1 - rx2) + 0.0001
    h = jnp.abs(ry1 - ry2) + 0.0001
    p = (w + h) / 2.0
    s = jnp.sqrt((w + p) * (h + p))
    rr = w / h

    inv_s = 1.0 / s
    r_max = jnp.maximum(rr, 1.0 / rr)
    alpha = -K * r_max
    gamma = inv_s * inv_s * 0.25

    x_ = (x1 + x2) * 0.5
    y_ = (y1 + y2) * 0.5
    w_ = jnp.abs(x1 - x2) + 0.0001
    h_ = jnp.abs(y1 - y2) + 0.0001
    p_ = (w_ + h_) * 0.5
    a_ = (w_ + p_) * (h_ + p_)
    ra = lax.rsqrt(a_)
    s_max = jnp.maximum(s * ra, a_ * ra * inv_s)
    penalty = jnp.exp(alpha * s_max + K)

    dx = x - x_
    dy = y - y_
    v = (dx * dx + dy * dy) * gamma
    han42 = 0.21 * _coswin(v) + 0.21
    han42 = jnp.where(v > 1.0, 0.0, han42)
    pw = s1 * penalty + han42

    m = jnp.max(pw, axis=1, keepdims=True)
    li = lax.broadcasted_iota(jnp.int32, (B, NC), 1)
    cand = jnp.where(pw == m, li, jnp.int32(NC))
    jm = jnp.min(cand, axis=1, keepdims=True)
    onehot = li == jm

    def sel(v):
        return jnp.sum(jnp.where(onehot, v, 0.0), axis=1, keepdims=True)

    out1_ref[:, 0:1] = sel(x1)
    out1_ref[:, 1:2] = sel(y1)
    out1_ref[:, 2:3] = sel(x2)
    out1_ref[:, 3:4] = sel(y2)
    out2_ref[:, 0:1] = sel(s1)


def tc_kernel(rois, rpn_t, sc_t, start, count):
    b0 = start // B
    grid = (count // B,)
    out1, out2 = pl.pallas_call(
        tc_body,
        grid=grid,
        in_specs=[
            pl.BlockSpec((B, 4), lambda i: (i + b0, 0)),
            pl.BlockSpec((1, B, NC), lambda i: (1, i + b0, 0)),
            pl.BlockSpec((1, B, NC), lambda i: (2, i + b0, 0)),
            pl.BlockSpec((1, B, NC), lambda i: (3, i + b0, 0)),
            pl.BlockSpec((1, B, NC), lambda i: (4, i + b0, 0)),
            pl.BlockSpec((B, 2, NC), lambda i: (i + b0, 0, 0)),
        ],
        out_specs=[
            pl.BlockSpec((B, 4), lambda i: (i, 0)),
            pl.BlockSpec((B, 1), lambda i: (i, 0)),
        ],
        out_shape=[
            jax.ShapeDtypeStruct((count, 4), jnp.float32),
            jax.ShapeDtypeStruct((count, 1), jnp.float32),
        ],
    )(rois, rpn_t, rpn_t, rpn_t, rpn_t, sc_t)
    return out1, out2


BE = 128


def _s1x_body(sc_ref, out_ref):
    out_ref[...] = sc_ref[:, 1, :]


def s1_extract(sc_t):
    return pl.pallas_call(
        _s1x_body,
        grid=(NSC // BE,),
        in_specs=[pl.BlockSpec((BE, 2, NC), lambda i: (i, 0, 0))],
        out_specs=pl.BlockSpec((BE, NC), lambda i: (i, 0)),
        out_shape=jax.ShapeDtypeStruct((NSC, NC), jnp.float32),
    )(sc_t)


@jax.jit
def kernel(rois, rpn_rois, scores):
    rpn_t = jnp.transpose(rpn_rois, (2, 0, 1))
    sc_t = jnp.transpose(scores, (0, 2, 1))
    o1sc, o2sc = _sc_trnms(rois, rpn_t, sc_t)
    t1, t2 = tc_kernel(rois, rpn_t, sc_t, NTC0, N - NTC0)
    out1 = jnp.concatenate([jnp.reshape(o1sc, (NSC, 4))[:NTC0], t1], axis=0)
    out2 = jnp.concatenate([o2sc[:NTC0, None], t2], axis=0)
    return out1, out2


# speedup vs baseline: 111.7339x; 1.0808x over previous
import functools

import jax
import jax.numpy as jnp
from jax import lax
from jax.experimental import pallas as pl
from jax.experimental.pallas import tpu as pltpu
from jax.experimental.pallas import tpu_sc as plsc

N = 5000
NCAND = 2000
NW = 32
NSC = 1536
RPW = NSC // NW
NTC0 = 1400

K = 0.055

_COS = (0.9999999999999956, -4.934802200543692, 4.058712126381724,
        -1.3352627683856102, 0.23533062721525902, -0.025806879372614583,
        0.0019295464417837132, -0.00010459814866408124, 4.268378688720986e-06)


def _rsqrt(x):
    i = lax.bitcast_convert_type(x, jnp.int32)
    i = jnp.int32(0x5F3759DF) - lax.shift_right_arithmetic(i, 1)
    y = lax.bitcast_convert_type(i, jnp.float32)
    for _ in range(3):
        y = y * (jnp.float32(1.5) - jnp.float32(0.5) * x * y * y)
    return y


def _coswin(v):
    acc = jnp.full_like(v, jnp.float32(_COS[-1]))
    for c in _COS[-2::-1]:
        acc = acc * v + jnp.float32(c)
    return acc


def _make_sc_call():
    mesh = plsc.VectorSubcoreMesh(core_axis_name="c", subcore_axis_name="s")

    @functools.partial(
        pl.kernel,
        mesh=mesh,
        compiler_params=pltpu.CompilerParams(
            needs_layout_passes=False, use_tc_tiling_on_sc=True),
        out_type=[
            jax.ShapeDtypeStruct((NSC * 4,), jnp.float32),
            jax.ShapeDtypeStruct((NSC,), jnp.float32),
        ],
        scratch_types=[
            pltpu.VMEM((RPW, 4), jnp.float32),
            pltpu.VMEM((4, 8, NCAND), jnp.float32),
            pltpu.VMEM((8, 2, NCAND), jnp.float32),
            pltpu.VMEM((RPW,), jnp.float32),
            pltpu.VMEM((RPW,), jnp.float32),
            pltpu.VMEM((RPW,), jnp.float32),
            pltpu.VMEM((RPW,), jnp.float32),
            pltpu.VMEM((RPW,), jnp.float32),
            pltpu.VMEM((RPW,), jnp.float32),
            pltpu.VMEM((RPW * 4,), jnp.float32),
            pltpu.VMEM((RPW,), jnp.float32),
        ],
    )
    def trnms(rois_hbm, rpn_hbm, s1_hbm, out1_hbm, out2_hbm,
              rois_v, pbuf, sbuf, cx, cy, cs, cis, cal, cga,
              o1v, o2v):
        wid = lax.axis_index("s") * 2 + lax.axis_index("c")
        base = wid * RPW

        iota = lax.iota(jnp.int32, 16)
        zeros = jnp.zeros((16,), jnp.int32)

        pltpu.sync_copy(rois_hbm.at[pl.ds(base, RPW), :], rois_v)

        for t in range(RPW // 16):
            li = iota + (16 * t)
            c0 = plsc.load_gather(rois_v, [li, zeros])
            c1 = plsc.load_gather(rois_v, [li, zeros + 1])
            c2 = plsc.load_gather(rois_v, [li, zeros + 2])
            c3 = plsc.load_gather(rois_v, [li, zeros + 3])
            x = (c0 + c2) * jnp.float32(0.5)
            y = (c1 + c3) * jnp.float32(0.5)
            w = jnp.abs(c0 - c2) + jnp.float32(1e-4)
            h = jnp.abs(c1 - c3) + jnp.float32(1e-4)
            p = (w + h) * jnp.float32(0.5)
            q = (w + p) * (h + p)
            rq = _rsqrt(q)
            s = q * rq
            r = w / h
            r_max = jnp.maximum(r, jnp.float32(1.0) / r)
            sl = pl.ds(16 * t, 16)
            cx[sl] = x
            cy[sl] = y
            cs[sl] = s
            cis[sl] = rq
            cal[sl] = jnp.float32(-K) * r_max
            cga[sl] = rq * rq * jnp.float32(0.25)

        for g in range(RPW // 8):
            r0 = base + g * 8
            pltpu.sync_copy(rpn_hbm.at[pl.ds(1, 4), pl.ds(r0, 8), :], pbuf)
            pltpu.sync_copy(s1_hbm.at[pl.ds(r0, 8), :, :], sbuf)

            def per_roi(r, carry):
                i = r + g * 8
                ii = zeros + i
                x = plsc.load_gather(cx, [ii])
                y = plsc.load_gather(cy, [ii])
                s = plsc.load_gather(cs, [ii])
                inv_s = plsc.load_gather(cis, [ii])
                alpha = plsc.load_gather(cal, [ii])
                gamma = plsc.load_gather(cga, [ii])

                def cand_step(jj, st):
                    bestv, besti = st
                    sl = pl.ds(jj * 16, 16)
                    c0 = pbuf[0, r, sl]
                    c1 = pbuf[1, r, sl]
                    c2 = pbuf[2, r, sl]
                    c3 = pbuf[3, r, sl]
                    s1 = sbuf[r, 1, sl]

                    x_ = (c0 + c2) * jnp.float32(0.5)
                    y_ = (c1 + c3) * jnp.float32(0.5)
                    w_ = jnp.abs(c0 - c2) + jnp.float32(1e-4)
                    h_ = jnp.abs(c1 - c3) + jnp.float32(1e-4)
                    p_ = (w_ + h_) * jnp.float32(0.5)
                    a_ = (w_ + p_) * (h_ + p_)
                    ra = _rsqrt(a_)
                    s_max = jnp.maximum(s * ra, a_ * ra * inv_s)
                    pen = jnp.exp(alpha * s_max + jnp.float32(K))

                    dx = x - x_
                    dy = y - y_
                    v = (dx * dx + dy * dy) * gamma
                    han42 = jnp.float32(0.21) * _coswin(v) + jnp.float32(0.21)
                    han42 = jnp.where(
                        v > jnp.float32(1.0), jnp.float32(0.0), han42)

                    score = s1 * pen + han42
                    jvec = iota + jj * 16
                    upd = score > bestv
                    bestv = jnp.where(upd, score, bestv)
                    besti = jnp.where(upd, jvec, besti)
                    return bestv, besti

                bestv, besti = lax.fori_loop(
                    0, NCAND // 16, cand_step,
                    (jnp.full((16,), -jnp.inf, jnp.float32),
                     jnp.zeros((16,), jnp.int32)))

                m = jnp.max(bestv)
                cand = jnp.where(bestv == m, besti, jnp.int32(1 << 30))
                minj = jnp.min(cand)
                jc = jnp.minimum(minj, jnp.int32(NCAND - 16))
                lane = minj - jc

                for f in range(4):
                    vals = pbuf[f, r, pl.ds(jc, 16)]
                    plsc.store_scatter(
                        o1v, [zeros + (i * 4 + f)], vals, mask=iota == lane)
                sv = sbuf[r, 1, pl.ds(jc, 16)]
                plsc.store_scatter(o2v, [zeros + i], sv, mask=iota == lane)
                return carry

            lax.fori_loop(0, 8, per_roi, jnp.int32(0))

        pltpu.sync_copy(o1v, out1_hbm.at[pl.ds(base * 4, RPW * 4)])
        pltpu.sync_copy(o2v, out2_hbm.at[pl.ds(base, RPW)])

    return trnms


_sc_trnms = _make_sc_call()


B = 200
PI = 3.141592653589793
NC = NCAND


def tc_body(rois_ref, rp1_ref, rp2_ref, rp3_ref, rp4_ref, sc_ref,
            out1_ref, out2_ref):
    x1 = rp1_ref[0]
    y1 = rp2_ref[0]
    x2 = rp3_ref[0]
    y2 = rp4_ref[0]
    s1 = sc_ref[:, 1, :]
    r = rois_ref[...]
    rx1 = r[:, 0:1]
    ry1 = r[:, 1:2]
    rx2 = r[:, 2:3]
    ry2 = r[:, 3:4]

    x = (rx1 + rx2) / 2.0
    y = (ry1 + ry2) / 2.0
    w = jnp.abs(rx1 - rx2) + 0.0001
    h = jnp.abs(ry1 - ry2) + 0.0001
    p = (w + h) / 2.0
    s = jnp.sqrt((w + p) * (h + p))
    rr = w / h

    inv_s = 1.0 / s
    r_max = jnp.maximum(rr, 1.0 / rr)
    alpha = -K * r_max
    gamma = inv_s * inv_s * 0.25

    x_ = (x1 + x2) * 0.5
    y_ = (y1 + y2) * 0.5
    w_ = jnp.abs(x1 - x2) + 0.0001
    h_ = jnp.abs(y1 - y2) + 0.0001
    p_ = (w_ + h_) * 0.5
    a_ = (w_ + p_) * (h_ + p_)
    ra = lax.rsqrt(a_)
    s_max = jnp.maximum(s * ra, a_ * ra * inv_s)
    penalty = jnp.exp(alpha * s_max + K)

    dx = x - x_
    dy = y - y_
    v = (dx * dx + dy * dy) * gamma
    han42 = 0.21 * _coswin(v) + 0.21
    han42 = jnp.where(v > 1.0, 0.0, han42)
    pw = s1 * penalty + han42

    m = jnp.max(pw, axis=1, keepdims=True)
    li = lax.broadcasted_iota(jnp.int32, (B, NC), 1)
    cand = jnp.where(pw == m, li, jnp.int32(NC))
    jm = jnp.min(cand, axis=1, keepdims=True)
    onehot = li == jm

    def sel(v):
        return jnp.sum(jnp.where(onehot, v, 0.0), axis=1, keepdims=True)

    out1_ref[:, 0:1] = sel(x1)
    out1_ref[:, 1:2] = sel(y1)
    out1_ref[:, 2:3] = sel(x2)
    out1_ref[:, 3:4] = sel(y2)
    out2_ref[:, 0:1] = sel(s1)


def tc_kernel(rois, rpn_t, sc_t, start, count):
    b0 = start // B
    grid = (count // B,)
    out1, out2 = pl.pallas_call(
        tc_body,
        grid=grid,
        in_specs=[
            pl.BlockSpec((B, 4), lambda i: (i + b0, 0)),
            pl.BlockSpec((1, B, NC), lambda i: (1, i + b0, 0)),
            pl.BlockSpec((1, B, NC), lambda i: (2, i + b0, 0)),
            pl.BlockSpec((1, B, NC), lambda i: (3, i + b0, 0)),
            pl.BlockSpec((1, B, NC), lambda i: (4, i + b0, 0)),
            pl.BlockSpec((B, 2, NC), lambda i: (i + b0, 0, 0)),
        ],
        out_specs=[
            pl.BlockSpec((B, 4), lambda i: (i, 0)),
            pl.BlockSpec((B, 1), lambda i: (i, 0)),
        ],
        out_shape=[
            jax.ShapeDtypeStruct((count, 4), jnp.float32),
            jax.ShapeDtypeStruct((count, 1), jnp.float32),
        ],
    )(rois, rpn_t, rpn_t, rpn_t, rpn_t, sc_t)
    return out1, out2


BE = 128


def _s1x_body(sc_ref, out_ref):
    out_ref[...] = sc_ref[:, 1, :]


def s1_extract(sc_t):
    return pl.pallas_call(
        _s1x_body,
        grid=(NSC // BE,),
        in_specs=[pl.BlockSpec((BE, 2, NC), lambda i: (i, 0, 0))],
        out_specs=pl.BlockSpec((BE, NC), lambda i: (i, 0)),
        out_shape=jax.ShapeDtypeStruct((NSC, NC), jnp.float32),
    )(sc_t)


@jax.jit
def kernel(rois, rpn_rois, scores):
    rpn_t = jnp.transpose(rpn_rois, (2, 0, 1))
    sc_t = jnp.transpose(scores, (0, 2, 1))
    o1sc, o2sc = _sc_trnms(rois, rpn_t, sc_t)
    t1, t2 = tc_kernel(rois, rpn_t, sc_t, NTC0, N - NTC0)
    out1 = jnp.concatenate([jnp.reshape(o1sc, (NSC, 4))[:NTC0], t1], axis=0)
    out2 = jnp.concatenate([o2sc[:NTC0, None], t2], axis=0)
    return out1, out2
